# baseline (speedup 1.0000x reference)
import functools

import jax
import jax.numpy as jnp
from jax import lax
from jax.experimental import pallas as pl
from jax.experimental.pallas import tpu as pltpu
from jax.experimental.pallas import tpu_sc as plsc

BATCH = 4096
SEQ = 200
DIM = 64
N = BATCH * SEQ
LANES = 16

NC = 2
NS = 16
NW = NC * NS
ROWS_PW = N // NW

SUB = 128
K = 8
CHUNK = K * SUB
NCHUNK = ROWS_PW // CHUNK

_mesh = plsc.VectorSubcoreMesh(
    core_axis_name="c", subcore_axis_name="s", num_cores=NC, num_subcores=NS
)


@functools.partial(
    pl.kernel,
    out_type=jax.ShapeDtypeStruct((N, DIM), jnp.float32),
    mesh=_mesh,
    compiler_params=pltpu.CompilerParams(use_tc_tiling_on_sc=False),
    scratch_types=[
        pltpu.VMEM((K, SUB), jnp.int32),
        pltpu.VMEM((CHUNK, DIM), jnp.float32),
        pltpu.VMEM((SEQ, DIM), jnp.float32),
        pltpu.SemaphoreType.DMA,
    ],
)
def _emb_lookup(idx_hbm, table_hbm, pos_hbm, out_hbm, idx_v, rows_v, pos_v, sem):
    wid = lax.axis_index("s") * NC + lax.axis_index("c")
    base_row = wid * ROWS_PW
    pltpu.sync_copy(pos_hbm, pos_v)

    def chunk_body(ci, p0):
        row0 = base_row + ci * CHUNK
        idx_row0 = pl.multiple_of(base_row // SUB + ci * K, 8)
        pltpu.sync_copy(idx_hbm.at[pl.ds(idx_row0, K)], idx_v)

        descs = [
            pltpu.async_copy(
                table_hbm.at[idx_v.at[j]], rows_v.at[pl.ds(j * SUB, SUB)], sem
            )
            for j in range(K)
        ]
        for d in descs:
            d.wait()

        def add_body(r, p):
            for c in range(DIM // LANES):
                pv = pos_v[p, pl.ds(c * LANES, LANES)]
                plsc.addupdate(rows_v.at[r, pl.ds(c * LANES, LANES)], pv)
            p = p + 1
            return lax.select(p == SEQ, 0, p)

        p_next = lax.fori_loop(0, CHUNK, add_body, p0)

        pltpu.sync_copy(rows_v, out_hbm.at[pl.ds(row0, CHUNK)])
        return p_next

    lax.fori_loop(0, NCHUNK, chunk_body, 0)


def kernel(words, word_table, pos_table):
    idx2d = words.astype(jnp.int32).reshape(N // SUB, SUB)
    out = _emb_lookup(idx2d, word_table, pos_table)
    return out.reshape(BATCH, SEQ, DIM)


# speedup vs baseline: 1.0715x; 1.0715x over previous
import functools

import jax
import jax.numpy as jnp
from jax import lax
from jax.experimental import pallas as pl
from jax.experimental.pallas import tpu as pltpu
from jax.experimental.pallas import tpu_sc as plsc

BATCH = 4096
SEQ = 200
DIM = 64
N = BATCH * SEQ
LANES = 16
GRP = DIM // LANES

NC = 2
NS = 16
NW = NC * NS
ROWS_PW = N // NW

SUB = 128
K = 4
CHUNK = K * SUB
NCHUNK = ROWS_PW // CHUNK
PAIRS = NCHUNK // 2
IDXROWS = ROWS_PW // SUB
UNROLL = 8

_mesh = plsc.VectorSubcoreMesh(
    core_axis_name="c", subcore_axis_name="s", num_cores=NC, num_subcores=NS
)


@functools.partial(
    pl.kernel,
    out_type=jax.ShapeDtypeStruct((N, DIM), jnp.float32),
    mesh=_mesh,
    compiler_params=pltpu.CompilerParams(use_tc_tiling_on_sc=False),
    scratch_types=[
        pltpu.VMEM((IDXROWS, SUB), jnp.int32),
        pltpu.VMEM((CHUNK, DIM), jnp.float32),
        pltpu.VMEM((CHUNK, DIM), jnp.float32),
        pltpu.VMEM((SEQ, DIM), jnp.float32),
        pltpu.SemaphoreType.DMA,
        pltpu.SemaphoreType.DMA,
        pltpu.SemaphoreType.DMA,
        pltpu.SemaphoreType.DMA,
    ],
)
def _emb_lookup(idx_hbm, table_hbm, pos_hbm, out_hbm,
                idx_v, rows0, rows1, pos_v, sem_g0, sem_g1, sem_s0, sem_s1):
    wid = lax.axis_index("s") * NC + lax.axis_index("c")
    base_row = wid * ROWS_PW
    base_idxrow = pl.multiple_of(wid * IDXROWS, 8)

    pltpu.sync_copy(pos_hbm, pos_v)
    pltpu.sync_copy(idx_hbm.at[pl.ds(base_idxrow, IDXROWS)], idx_v)

    def fire_gathers(ci, rows_ref, sem):
        for j in range(K):
            pltpu.async_copy(
                table_hbm.at[idx_v.at[ci * K + j]],
                rows_ref.at[pl.ds(j * SUB, SUB)],
                sem,
            )

    def drain_gathers(rows_ref, sem):
        pltpu.make_async_copy(table_hbm.at[pl.ds(0, CHUNK)], rows_ref, sem).wait()

    def fire_store(ci, rows_ref, sem):
        row0 = pl.multiple_of(base_row + ci * CHUNK, 8)
        pltpu.async_copy(rows_ref, out_hbm.at[pl.ds(row0, CHUNK)], sem)

    def drain_store(rows_ref, sem):
        pltpu.make_async_copy(rows_ref, out_hbm.at[pl.ds(base_row, CHUNK)], sem).wait()

    def add_pos(ci, rows_ref):
        p0 = lax.rem(ci * CHUNK, SEQ)

        def body(rr, p):
            for u in range(UNROLL):
                r = rr * UNROLL + u
                for c in range(GRP):
                    pv = pos_v[p, pl.ds(c * LANES, LANES)]
                    plsc.addupdate(rows_ref.at[r, pl.ds(c * LANES, LANES)], pv)
                p = p + 1
                p = lax.select(p == SEQ, 0, p)
            return p

        lax.fori_loop(0, CHUNK // UNROLL, body, p0)

    fire_gathers(0, rows0, sem_g0)

    def pair_body(t, _):
        a = 2 * t
        b = 2 * t + 1

        @pl.when(t > 0)
        def _():
            drain_store(rows1, sem_s1)

        fire_gathers(b, rows1, sem_g1)

        drain_gathers(rows0, sem_g0)
        add_pos(a, rows0)
        fire_store(a, rows0, sem_s0)

        drain_gathers(rows1, sem_g1)
        add_pos(b, rows1)
        fire_store(b, rows1, sem_s1)

        @pl.when(t < PAIRS - 1)
        def _():
            drain_store(rows0, sem_s0)
            fire_gathers(2 * t + 2, rows0, sem_g0)

        return 0

    lax.fori_loop(0, PAIRS, pair_body, 0)
    drain_store(rows0, sem_s0)
    drain_store(rows1, sem_s1)


def kernel(words, word_table, pos_table):
    idx2d = words.astype(jnp.int32).reshape(N // SUB, SUB)
    out = _emb_lookup(idx2d, word_table, pos_table)
    return out.reshape(BATCH, SEQ, DIM)
